# Initial kernel scaffold; baseline (speedup 1.0000x reference)
import jax
import jax.numpy as jnp
from jax.experimental import pallas as pl


def kernel(pos, edge_index, edge_weight):
    raise NotImplementedError("write your pallas kernel here")


# speedup vs baseline: 7.9580x; 7.9580x over previous
import functools

import jax
import jax.numpy as jnp
from jax import lax
from jax.experimental import pallas as pl
from jax.experimental.pallas import tpu as pltpu
from jax.experimental.pallas import tpu_sc as plsc

_NUM_CORES = 2
_NUM_SUBCORES = 16
_NUM_WORKERS = _NUM_CORES * _NUM_SUBCORES
_LANES = 16


def _pick_chunk(per_worker: int) -> int:
    for c in range(min(4000, per_worker), 7, -8):
        if per_worker % c == 0:
            return c
    return per_worker


def _sc_body(px_hbm, py_hbm, pz_hbm, row_hbm, col_hbm, ew_hbm, out_hbm,
             idxr_v, idxc_v, ew_v, out_v,
             xr_v, yr_v, zr_v, xc_v, yc_v, zc_v, sem_r, sem_c,
             *, per_worker: int, chunk: int):
    wid = lax.axis_index("s") * _NUM_CORES + lax.axis_index("c")
    base = wid * per_worker
    n_chunks = per_worker // chunk
    n_vec = chunk // _LANES

    lane_iota = lax.iota(jnp.int32, _LANES)
    one = jnp.full((_LANES,), 1, jnp.int32)
    two = jnp.full((_LANES,), 2, jnp.int32)
    three = jnp.full((_LANES,), 3, jnp.int32)
    half = jnp.float32(0.5)
    threehalf = jnp.float32(1.5)
    magic = jnp.int32(0x5F3759DF)

    def chunk_body(k, _):
        off = base + k * chunk
        pltpu.sync_copy(row_hbm.at[pl.ds(off, chunk)], idxr_v)
        pltpu.sync_copy(col_hbm.at[pl.ds(off, chunk)], idxc_v)
        pltpu.sync_copy(ew_hbm.at[pl.ds(off, chunk)], ew_v)
        cps = [
            pltpu.async_copy(px_hbm.at[idxr_v], xr_v, sem_r),
            pltpu.async_copy(py_hbm.at[idxr_v], yr_v, sem_r),
            pltpu.async_copy(pz_hbm.at[idxr_v], zr_v, sem_r),
            pltpu.async_copy(px_hbm.at[idxc_v], xc_v, sem_c),
            pltpu.async_copy(py_hbm.at[idxc_v], yc_v, sem_c),
            pltpu.async_copy(pz_hbm.at[idxc_v], zc_v, sem_c),
        ]
        for cp in cps:
            cp.wait()

        def vec_body(i, _):
            e0 = i * _LANES
            f0 = (lane_iota + e0) * 4
            f1 = f0 + one
            f2 = f0 + two
            f3 = f0 + three
            rx = xr_v[pl.ds(e0, _LANES)]
            ry = yr_v[pl.ds(e0, _LANES)]
            rz = zr_v[pl.ds(e0, _LANES)]
            cx = xc_v[pl.ds(e0, _LANES)]
            cy = yc_v[pl.ds(e0, _LANES)]
            cz = zc_v[pl.ds(e0, _LANES)]
            dx = cx - rx
            dy = cy - ry
            dz = cz - rz
            s = dx * dx + dy * dy + dz * dz
            s_bits = lax.bitcast_convert_type(s, jnp.int32)
            y = lax.bitcast_convert_type(magic - (s_bits >> 1), jnp.float32)
            xh = s * half
            y = y * (threehalf - xh * y * y)
            y = y * (threehalf - xh * y * y)
            y = y * (threehalf - xh * y * y)
            h = y * half
            plsc.store_scatter(out_v, [f0], dx * h + half)
            plsc.store_scatter(out_v, [f1], dy * h + half)
            plsc.store_scatter(out_v, [f2], dz * h + half)
            plsc.store_scatter(out_v, [f3], ew_v[pl.ds(e0, _LANES)])
            return _

        lax.fori_loop(0, n_vec, vec_body, None)
        pltpu.sync_copy(out_v, out_hbm.at[pl.ds(off * 4, chunk * 4)])
        return _

    lax.fori_loop(0, n_chunks, chunk_body, None)


@functools.cache
def _build(n_edges: int):
    per_worker = n_edges // _NUM_WORKERS
    chunk = _pick_chunk(per_worker)
    mesh = plsc.VectorSubcoreMesh(core_axis_name="c", subcore_axis_name="s",
                                  num_cores=_NUM_CORES,
                                  num_subcores=_NUM_SUBCORES)
    return pl.kernel(
        functools.partial(_sc_body, per_worker=per_worker, chunk=chunk),
        out_type=jax.ShapeDtypeStruct((n_edges * 4,), jnp.float32),
        mesh=mesh,
        scratch_types=[
            pltpu.VMEM((chunk,), jnp.int32),
            pltpu.VMEM((chunk,), jnp.int32),
            pltpu.VMEM((chunk,), jnp.float32),
            pltpu.VMEM((chunk * 4,), jnp.float32),
            pltpu.VMEM((chunk,), jnp.float32),
            pltpu.VMEM((chunk,), jnp.float32),
            pltpu.VMEM((chunk,), jnp.float32),
            pltpu.VMEM((chunk,), jnp.float32),
            pltpu.VMEM((chunk,), jnp.float32),
            pltpu.VMEM((chunk,), jnp.float32),
            pltpu.SemaphoreType.DMA,
            pltpu.SemaphoreType.DMA,
        ],
        compiler_params=pltpu.CompilerParams(needs_layout_passes=False),
    )


def kernel(pos, edge_index, edge_weight):
    n_edges = edge_weight.shape[0]
    row = edge_index[0].astype(jnp.int32)
    col = edge_index[1].astype(jnp.int32)
    posf = pos.astype(jnp.float32)
    px, py, pz = posf[:, 0], posf[:, 1], posf[:, 2]
    flat = _build(n_edges)(px, py, pz, row, col,
                           edge_weight.astype(jnp.float32))
    return flat.reshape(n_edges, 4)
